# baseline (speedup 1.0000x reference)
import functools

import jax
import jax.numpy as jnp
import numpy as np
from jax import lax
from jax.experimental import pallas as pl
from jax.experimental.pallas import tpu as pltpu
from jax.experimental.pallas import tpu_sc as plsc

_INDEX_NP = np.array(
    [0, 7777, 15554, 23331, 31108, 38885, 46662, 54439, 62216, 69993,
     77770, 85547, 93324, 101101, 108878, 116655, 124432, 132209, 139986,
     147763, 155540, 163317, 171094, 178871, 186648, 194425, 202202,
     209979, 217756, 225533, 233310, 241087, 248864, 256641, 264418,
     272195, 279972, 287749, 295526, 303303, 311080, 318857, 326634,
     334411, 342188, 349965, 357742, 365519, 373296, 381073, 388850,
     396627, 404404, 412181, 419958, 427735, 435512, 443289, 451066,
     458843, 466620, 474397, 482174, 489951], dtype=np.int32)

_B = 64
_D = 64
_RPW = 8
_NACT = _B // _RPW

_mesh = plsc.VectorSubcoreMesh(core_axis_name="c", subcore_axis_name="s")


@functools.partial(
    pl.kernel,
    mesh=_mesh,
    out_type=jax.ShapeDtypeStruct((_B, _D), jnp.float32),
    scratch_types=[
        pltpu.VMEM((_RPW,), jnp.int32),
        pltpu.VMEM((_RPW, _D), jnp.float32),
        pltpu.SemaphoreType.DMA,
    ],
    compiler_params=pltpu.CompilerParams(use_tc_tiling_on_sc=False),
)
def _gather_rows(table_hbm, idx_hbm, out_hbm, idx_v, rows_v, sem):
    wid = lax.axis_index("s") * 2 + lax.axis_index("c")

    @pl.when(wid < _NACT)
    def _():
        base = wid * _RPW
        pltpu.sync_copy(idx_hbm.at[pl.ds(base, _RPW)], idx_v)
        pltpu.async_copy(table_hbm.at[idx_v], rows_v, sem).wait()
        pltpu.sync_copy(rows_v, out_hbm.at[pl.ds(base, _RPW)])


def kernel(x):
    idx = jnp.asarray(_INDEX_NP)
    return _gather_rows(x, idx)


# speedup vs baseline: 1.7408x; 1.7408x over previous
import functools

import jax
import jax.numpy as jnp
import numpy as np
from jax import lax
from jax.experimental import pallas as pl
from jax.experimental.pallas import tpu as pltpu
from jax.experimental.pallas import tpu_sc as plsc

_INDEX_NP = np.array(
    [0, 7777, 15554, 23331, 31108, 38885, 46662, 54439, 62216, 69993,
     77770, 85547, 93324, 101101, 108878, 116655, 124432, 132209, 139986,
     147763, 155540, 163317, 171094, 178871, 186648, 194425, 202202,
     209979, 217756, 225533, 233310, 241087, 248864, 256641, 264418,
     272195, 279972, 287749, 295526, 303303, 311080, 318857, 326634,
     334411, 342188, 349965, 357742, 365519, 373296, 381073, 388850,
     396627, 404404, 412181, 419958, 427735, 435512, 443289, 451066,
     458843, 466620, 474397, 482174, 489951], dtype=np.int32)
_ROWS = [int(v) for v in _INDEX_NP]

_B = 64
_D = 64
_RPW = 8
_NACT = _B // _RPW

_mesh = plsc.VectorSubcoreMesh(core_axis_name="c", subcore_axis_name="s")


@functools.partial(
    pl.kernel,
    mesh=_mesh,
    out_type=jax.ShapeDtypeStruct((_B, _D), jnp.float32),
    scratch_types=[
        pltpu.VMEM((_RPW, _D), jnp.float32),
        pltpu.SemaphoreType.DMA,
    ],
)
def _gather_rows(table_hbm, out_hbm, rows_v, sem):
    wid = lax.axis_index("s") * 2 + lax.axis_index("c")

    for k in range(_NACT):
        @pl.when(wid == k)
        def _(k=k):
            copies = [
                pltpu.async_copy(
                    table_hbm.at[pl.ds(_ROWS[k * _RPW + j], 1)],
                    rows_v.at[pl.ds(j, 1)],
                    sem,
                )
                for j in range(_RPW)
            ]
            for c in copies:
                c.wait()
            pltpu.sync_copy(rows_v, out_hbm.at[pl.ds(k * _RPW, _RPW)])


def kernel(x):
    return _gather_rows(x)
